# Initial kernel scaffold; baseline (speedup 1.0000x reference)
import jax
import jax.numpy as jnp
from jax.experimental import pallas as pl


def kernel(inputs, time_encodings):
    raise NotImplementedError("write your pallas kernel here")


# speedup vs baseline: 2.8577x; 2.8577x over previous
import functools

import jax
import jax.numpy as jnp
from jax import lax
from jax.experimental import pallas as pl
from jax.experimental.pallas import tpu as pltpu
from jax.experimental.pallas import tpu_sc as plsc

_T1 = 1001
_D = 128
_B = 16384
_H = 20
_FLAT = _B * _H
_G = 128

_info = plsc.get_sparse_core_info()
_NC = _info.num_cores
_NS = _info.num_subcores
_NW = _NC * _NS
_GPW = _FLAT // (_NW * _G)

_mesh = plsc.VectorSubcoreMesh(core_axis_name="c", subcore_axis_name="s")


@functools.partial(
    pl.kernel,
    mesh=_mesh,
    out_type=jax.ShapeDtypeStruct((_FLAT, _D), jnp.float32),
    scratch_types=[
        pltpu.VMEM((_GPW, _G), jnp.int32),
        pltpu.VMEM((_G, _D), jnp.float32),
        pltpu.SemaphoreType.DMA,
    ],
)
def _sc_gather(idx_hbm, table_hbm, out_hbm, idx_v, rows_v, sem):
    wid = lax.axis_index("s") * _NC + lax.axis_index("c")
    pltpu.sync_copy(idx_hbm.at[pl.ds(wid * _GPW, _GPW)], idx_v)

    def body(j, carry):
        pltpu.async_copy(table_hbm.at[idx_v.at[j]], rows_v, sem).wait()
        pltpu.sync_copy(rows_v, out_hbm.at[pl.ds((wid * _GPW + j) * _G, _G)])
        return carry

    lax.fori_loop(0, _GPW, body, 0)


def kernel(inputs, time_encodings):
    idx2d = inputs.reshape(_NW * _GPW, _G)
    out = _sc_gather(idx2d, time_encodings)
    return out.reshape(_B, _H, _D)


# speedup vs baseline: 2.9659x; 1.0379x over previous
import functools

import jax
import jax.numpy as jnp
from jax import lax
from jax.experimental import pallas as pl
from jax.experimental.pallas import tpu as pltpu
from jax.experimental.pallas import tpu_sc as plsc

_T1 = 1001
_D = 128
_B = 16384
_H = 20
_FLAT = _B * _H
_G = 128

_info = plsc.get_sparse_core_info()
_NC = _info.num_cores
_NS = _info.num_subcores
_NW = _NC * _NS
_GPW = _FLAT // (_NW * _G)

_mesh = plsc.VectorSubcoreMesh(core_axis_name="c", subcore_axis_name="s")


_NBUF = 5
_LEAD = 3


@functools.partial(
    pl.kernel,
    mesh=_mesh,
    out_type=jax.ShapeDtypeStruct((_FLAT, _D), jnp.float32),
    scratch_types=[
        pltpu.VMEM((_GPW, _G), jnp.int32),
        pltpu.VMEM((_NBUF, _G, _D), jnp.float32),
        pltpu.SemaphoreType.DMA((_NBUF,)),
        pltpu.SemaphoreType.DMA((_NBUF,)),
    ],
)
def _sc_gather(idx_hbm, table_hbm, out_hbm, idx_v, rows_v, gsem, ssem):
    wid = lax.axis_index("s") * _NC + lax.axis_index("c")
    base = wid * _GPW
    pltpu.sync_copy(idx_hbm.at[pl.ds(base, _GPW)], idx_v)

    def fire_gather(slot, g):
        pltpu.async_copy(
            table_hbm.at[idx_v.at[g]], rows_v.at[slot], gsem.at[slot])

    def wait_gather(slot, g):
        pltpu.make_async_copy(
            table_hbm.at[idx_v.at[g]], rows_v.at[slot], gsem.at[slot]).wait()

    def fire_store(slot, g):
        pltpu.async_copy(
            rows_v.at[slot], out_hbm.at[pl.ds((base + g) * _G, _G)],
            ssem.at[slot])

    def wait_store(slot, g):
        pltpu.make_async_copy(
            rows_v.at[slot], out_hbm.at[pl.ds((base + g) * _G, _G)],
            ssem.at[slot]).wait()

    def outer(o, carry):
        for b in range(_NBUF):
            j = o * _NBUF + b

            @pl.when(j >= _NBUF)
            def _():
                wait_store(b, j - _NBUF)

            fire_gather(b, j)
            wait_gather(b, j)
            fire_store(b, j)
        return carry

    lax.fori_loop(0, _GPW // _NBUF, outer, 0)

    for g in range(_GPW - _NBUF, _GPW):
        wait_store(g % _NBUF, g)


def kernel(inputs, time_encodings):
    idx2d = inputs.reshape(_NW * _GPW, _G)
    out = _sc_gather(idx2d, time_encodings)
    return out.reshape(_B, _H, _D)


# speedup vs baseline: 3.0008x; 1.0117x over previous
import functools

import jax
import jax.numpy as jnp
from jax import lax
from jax.experimental import pallas as pl
from jax.experimental.pallas import tpu as pltpu
from jax.experimental.pallas import tpu_sc as plsc

_T1 = 1001
_D = 128
_B = 16384
_H = 20
_FLAT = _B * _H
_G = 128

_info = plsc.get_sparse_core_info()
_NC = _info.num_cores
_NS = _info.num_subcores
_NW = _NC * _NS
_GPW = _FLAT // (_NW * _G)

_mesh = plsc.VectorSubcoreMesh(core_axis_name="c", subcore_axis_name="s")


_NBUF = 5
_LEAD = 3


@functools.partial(
    pl.kernel,
    mesh=_mesh,
    out_type=jax.ShapeDtypeStruct((_FLAT, _D), jnp.float32),
    scratch_types=[
        pltpu.VMEM((_GPW, _G), jnp.int32),
        pltpu.VMEM((_NBUF, _G, _D), jnp.float32),
        pltpu.SemaphoreType.DMA((_NBUF,)),
        pltpu.SemaphoreType.DMA((_NBUF,)),
    ],
)
def _sc_gather(idx_hbm, table_hbm, out_hbm, idx_v, rows_v, gsem, ssem):
    wid = lax.axis_index("s") * _NC + lax.axis_index("c")
    base = wid * _GPW
    pltpu.sync_copy(idx_hbm.at[pl.ds(base, _GPW)], idx_v)

    def fire_gather(slot, g):
        return pltpu.async_copy(
            table_hbm.at[idx_v.at[g]], rows_v.at[slot], gsem.at[slot])

    def wait_gather(slot, g):
        pltpu.make_async_copy(
            table_hbm.at[idx_v.at[g]], rows_v.at[slot], gsem.at[slot]).wait()

    def fire_store(slot, g):
        pltpu.async_copy(
            rows_v.at[slot], out_hbm.at[pl.ds((base + g) * _G, _G)],
            ssem.at[slot])

    def wait_store(slot, g):
        pltpu.make_async_copy(
            rows_v.at[slot], out_hbm.at[pl.ds((base + g) * _G, _G)],
            ssem.at[slot]).wait()

    def outer(o, carry):
        handles = []
        for b in range(_NBUF):
            j = o * _NBUF + b

            @pl.when(j >= _NBUF)
            def _():
                wait_store(b, j - _NBUF)

            handles.append(fire_gather(b, j))
        for b in range(_NBUF):
            j = o * _NBUF + b
            handles[b].wait()
            fire_store(b, j)
        return carry

    lax.fori_loop(0, _GPW // _NBUF, outer, 0)

    for g in range(_GPW - _NBUF, _GPW):
        wait_store(g % _NBUF, g)


def kernel(inputs, time_encodings):
    idx2d = inputs.reshape(_NW * _GPW, _G)
    out = _sc_gather(idx2d, time_encodings)
    return out.reshape(_B, _H, _D)


# speedup vs baseline: 3.8624x; 1.2872x over previous
import functools

import jax
import jax.numpy as jnp
from jax import lax
from jax.experimental import pallas as pl
from jax.experimental.pallas import tpu as pltpu
from jax.experimental.pallas import tpu_sc as plsc

_T1 = 1001
_D = 128
_B = 16384
_H = 20
_FLAT = _B * _H
_G = 128

_info = plsc.get_sparse_core_info()
_NC = _info.num_cores
_NS = _info.num_subcores
_NW = _NC * _NS
_GPW = _FLAT // (_NW * _G)

_mesh = plsc.VectorSubcoreMesh(core_axis_name="c", subcore_axis_name="s")


_NBUF = 5
_LEAD = 3


@functools.partial(
    pl.kernel,
    mesh=_mesh,
    out_type=jax.ShapeDtypeStruct((_FLAT, _D), jnp.float32),
    scratch_types=[
        pltpu.VMEM((_GPW, _G), jnp.int32),
        pltpu.VMEM((_NBUF, _G, _D), jnp.float32),
        pltpu.VMEM_SHARED((_T1, _D), jnp.float32),
        pltpu.SemaphoreType.DMA((_NBUF,)),
        pltpu.SemaphoreType.DMA((_NBUF,)),
    ],
)
def _sc_gather(idx_hbm, table_hbm, out_hbm, idx_v, rows_v, table_sp,
               gsem, ssem):
    sid = lax.axis_index("s")
    wid = sid * _NC + lax.axis_index("c")
    base = wid * _GPW

    @pl.when(sid == 0)
    def _():
        pltpu.sync_copy(table_hbm, table_sp)

    pltpu.sync_copy(idx_hbm.at[pl.ds(base, _GPW)], idx_v)
    plsc.subcore_barrier()

    def fire_gather(slot, g):
        return pltpu.async_copy(
            table_sp.at[idx_v.at[g]], rows_v.at[slot], gsem.at[slot])

    def fire_store(slot, g):
        pltpu.async_copy(
            rows_v.at[slot], out_hbm.at[pl.ds((base + g) * _G, _G)],
            ssem.at[slot])

    def wait_store(slot, g):
        pltpu.make_async_copy(
            rows_v.at[slot], out_hbm.at[pl.ds((base + g) * _G, _G)],
            ssem.at[slot]).wait()

    def outer(o, carry):
        handles = []
        for b in range(_NBUF):
            j = o * _NBUF + b

            @pl.when(j >= _NBUF)
            def _():
                wait_store(b, j - _NBUF)

            handles.append(fire_gather(b, j))
        for b in range(_NBUF):
            j = o * _NBUF + b
            handles[b].wait()
            fire_store(b, j)
        return carry

    lax.fori_loop(0, _GPW // _NBUF, outer, 0)

    for g in range(_GPW - _NBUF, _GPW):
        wait_store(g % _NBUF, g)


def kernel(inputs, time_encodings):
    idx2d = inputs.reshape(_NW * _GPW, _G)
    out = _sc_gather(idx2d, time_encodings)
    return out.reshape(_B, _H, _D)


# speedup vs baseline: 6.2014x; 1.6056x over previous
import functools

import jax
import jax.numpy as jnp
from jax import lax
from jax.experimental import pallas as pl
from jax.experimental.pallas import tpu as pltpu
from jax.experimental.pallas import tpu_sc as plsc

_T1 = 1001
_D = 128
_B = 16384
_H = 20

_info = plsc.get_sparse_core_info()
_NC = _info.num_cores
_NS = _info.num_subcores
_NW = _NC * _NS
_RPW = _B // _NW

_NBR = 16
_GI = _NBR * _H
_GPW = _RPW // _NBR
_NBUF = 2

_mesh = plsc.VectorSubcoreMesh(core_axis_name="c", subcore_axis_name="s")


@functools.partial(
    pl.kernel,
    mesh=_mesh,
    out_type=jax.ShapeDtypeStruct((_B, _H, _D), jnp.float32),
    scratch_types=[
        pltpu.VMEM((_GPW * _GI,), jnp.int32),
        pltpu.VMEM((_NBUF, _GI, _D), jnp.float32),
        pltpu.VMEM_SHARED((_T1, _D), jnp.float32),
        pltpu.SemaphoreType.DMA((_NBUF,)),
        pltpu.SemaphoreType.DMA((_NBUF,)),
    ],
)
def _sc_gather(idx_hbm, table_hbm, out_hbm, idx_v, rows_v, table_sp,
               gsem, ssem):
    sid = lax.axis_index("s")
    wid = sid * _NC + lax.axis_index("c")
    base = wid * _RPW

    @pl.when(sid == 0)
    def _():
        pltpu.sync_copy(table_hbm, table_sp)

    pltpu.sync_copy(idx_hbm.at[pl.ds(wid * _GPW * _GI, _GPW * _GI)], idx_v)
    plsc.subcore_barrier()

    def fire_gather(slot, g):
        return pltpu.async_copy(
            table_sp.at[idx_v.at[pl.ds(g * _GI, _GI)]], rows_v.at[slot],
            gsem.at[slot])

    def fire_store(slot, g):
        for r in range(_NBR):
            pltpu.async_copy(
                rows_v.at[slot, pl.ds(r * _H, _H)],
                out_hbm.at[base + g * _NBR + r], ssem.at[slot])

    def wait_store(slot, g):
        for r in range(_NBR):
            pltpu.make_async_copy(
                rows_v.at[slot, pl.ds(r * _H, _H)],
                out_hbm.at[base + g * _NBR + r], ssem.at[slot]).wait()

    def outer(o, carry):
        handles = []
        for b in range(_NBUF):
            g = o * _NBUF + b

            @pl.when(g >= _NBUF)
            def _():
                wait_store(b, g - _NBUF)

            handles.append(fire_gather(b, g))
        for b in range(_NBUF):
            g = o * _NBUF + b
            handles[b].wait()
            fire_store(b, g)
        return carry

    lax.fori_loop(0, _GPW // _NBUF, outer, 0)

    for g in range(_GPW - _NBUF, _GPW):
        wait_store(g % _NBUF, g)


def kernel(inputs, time_encodings):
    return _sc_gather(inputs.reshape(-1), time_encodings)


# speedup vs baseline: 6.2091x; 1.0012x over previous
import functools

import jax
import jax.numpy as jnp
from jax import lax
from jax.experimental import pallas as pl
from jax.experimental.pallas import tpu as pltpu
from jax.experimental.pallas import tpu_sc as plsc

_T1 = 1001
_D = 128
_B = 16384
_H = 20

_info = plsc.get_sparse_core_info()
_NC = _info.num_cores
_NS = _info.num_subcores
_NW = _NC * _NS
_RPW = _B // _NW

_NBR = 16
_GI = _NBR * _H
_GPW = _RPW // _NBR
_NBUF = 2

_mesh = plsc.VectorSubcoreMesh(core_axis_name="c", subcore_axis_name="s")


@functools.partial(
    pl.kernel,
    mesh=_mesh,
    out_type=jax.ShapeDtypeStruct((_B, _H, _D), jnp.float32),
    compiler_params=pltpu.CompilerParams(use_tc_tiling_on_sc=True),
    scratch_types=[
        pltpu.VMEM((_GPW * _GI,), jnp.int32),
        pltpu.VMEM((_NBUF, _GI, _D), jnp.float32),
        pltpu.VMEM_SHARED((_T1, _D), jnp.float32),
        pltpu.SemaphoreType.DMA((_NBUF,)),
        pltpu.SemaphoreType.DMA((_NBUF,)),
    ],
)
def _sc_gather(idx_hbm, table_hbm, out_hbm, idx_v, rows_v, table_sp,
               gsem, ssem):
    sid = lax.axis_index("s")
    wid = sid * _NC + lax.axis_index("c")
    base = wid * _RPW

    @pl.when(sid == 0)
    def _():
        pltpu.sync_copy(table_hbm, table_sp)

    pltpu.sync_copy(idx_hbm.at[pl.ds(wid * _GPW * _GI, _GPW * _GI)], idx_v)
    plsc.subcore_barrier()

    def fire_gather(slot, g):
        return pltpu.async_copy(
            table_sp.at[idx_v.at[pl.ds(g * _GI, _GI)]], rows_v.at[slot],
            gsem.at[slot])

    def fire_store(slot, g):
        for r in range(_NBR):
            pltpu.async_copy(
                rows_v.at[slot, pl.ds(r * _H, _H)],
                out_hbm.at[base + g * _NBR + r], ssem.at[slot])

    def wait_store(slot, g):
        for r in range(_NBR):
            pltpu.make_async_copy(
                rows_v.at[slot, pl.ds(r * _H, _H)],
                out_hbm.at[base + g * _NBR + r], ssem.at[slot]).wait()

    def outer(o, carry):
        handles = []
        for b in range(_NBUF):
            g = o * _NBUF + b

            @pl.when(g >= _NBUF)
            def _():
                wait_store(b, g - _NBUF)

            handles.append(fire_gather(b, g))
        for b in range(_NBUF):
            g = o * _NBUF + b
            handles[b].wait()
            fire_store(b, g)
        return carry

    lax.fori_loop(0, _GPW // _NBUF, outer, 0)

    for g in range(_GPW - _NBUF, _GPW):
        wait_store(g % _NBUF, g)


def kernel(inputs, time_encodings):
    return _sc_gather(inputs.reshape(-1), time_encodings)


# speedup vs baseline: 17.8440x; 2.8738x over previous
import functools

import jax
import jax.numpy as jnp
from jax import lax
from jax.experimental import pallas as pl
from jax.experimental.pallas import tpu as pltpu
from jax.experimental.pallas import tpu_sc as plsc

_T1 = 1001
_D = 128
_B = 16384
_H = 20

_info = plsc.get_sparse_core_info()
_NC = _info.num_cores
_NS = _info.num_subcores
_NW = _NC * _NS
_RPW = _B // _NW

_CH = 128
_NCH = _RPW // _CH
_GPW = _H * _NCH
_NBUF = 5

_mesh = plsc.VectorSubcoreMesh(core_axis_name="c", subcore_axis_name="s")


@functools.partial(
    pl.kernel,
    mesh=_mesh,
    out_type=jax.ShapeDtypeStruct((_H, _B, _D), jnp.float32),
    scratch_types=[
        pltpu.VMEM((_H, _RPW), jnp.int32),
        pltpu.VMEM((_NBUF, _CH, _D), jnp.float32),
        pltpu.VMEM_SHARED((_T1, _D), jnp.float32),
        pltpu.SemaphoreType.DMA((_NBUF,)),
        pltpu.SemaphoreType.DMA((_NBUF,)),
    ],
)
def _sc_gather(idx_hbm, table_hbm, out_hbm, idx_v, rows_v, table_sp,
               gsem, ssem):
    sid = lax.axis_index("s")
    wid = sid * _NC + lax.axis_index("c")
    base = wid * _RPW

    @pl.when(sid == 0)
    def _():
        pltpu.sync_copy(table_hbm, table_sp)

    pltpu.sync_copy(idx_hbm.at[pl.ds(0, _H), pl.ds(base, _RPW)], idx_v)
    plsc.subcore_barrier()

    def fire_gather(slot, g):
        h = g // _NCH
        c = g % _NCH
        return pltpu.async_copy(
            table_sp.at[idx_v.at[h, pl.ds(c * _CH, _CH)]], rows_v.at[slot],
            gsem.at[slot])

    def fire_store(slot, g):
        h = g // _NCH
        c = g % _NCH
        pltpu.async_copy(
            rows_v.at[slot], out_hbm.at[h, pl.ds(base + c * _CH, _CH)],
            ssem.at[slot])

    def wait_store(slot, g):
        h = g // _NCH
        c = g % _NCH
        pltpu.make_async_copy(
            rows_v.at[slot], out_hbm.at[h, pl.ds(base + c * _CH, _CH)],
            ssem.at[slot]).wait()

    def outer(o, carry):
        handles = []
        for b in range(_NBUF):
            g = o * _NBUF + b

            @pl.when(g >= _NBUF)
            def _():
                wait_store(b, g - _NBUF)

            handles.append(fire_gather(b, g))
        for b in range(_NBUF):
            g = o * _NBUF + b
            handles[b].wait()
            fire_store(b, g)
        return carry

    lax.fori_loop(0, _GPW // _NBUF, outer, 0)

    for g in range(_GPW - _NBUF, _GPW):
        wait_store(g % _NBUF, g)


def kernel(inputs, time_encodings):
    out_hbd = _sc_gather(inputs.T, time_encodings)
    return jnp.transpose(out_hbd, (1, 0, 2))
